# Initial kernel scaffold; baseline (speedup 1.0000x reference)
import jax
import jax.numpy as jnp
from jax.experimental import pallas as pl


def kernel(input_seq_batch, table):
    raise NotImplementedError("write your pallas kernel here")


# speedup vs baseline: 9.0343x; 9.0343x over previous
import jax
import jax.numpy as jnp
from jax import lax
from jax.experimental import pallas as pl
from jax.experimental.pallas import tpu as pltpu
from jax.experimental.pallas import tpu_sc as plsc

BATCH = 4096
SEQ = 50
DIM = 64
NC = 2
NS = 16
NW = NC * NS
BPW = BATCH // NW
NBUF = 4
LANES = 16
CHUNKS = DIM // LANES


def _sc_body(idx_hbm, table_hbm, out_hbm, idx_v, rows_v, out_v, *sems):
    wid = lax.axis_index("s") * NC + lax.axis_index("c")
    base = wid * BPW
    pltpu.sync_copy(idx_hbm.at[pl.ds(base, BPW)], idx_v)

    def issue(r, b):
        pltpu.async_copy(table_hbm.at[idx_v.at[r]], rows_v.at[b], sems[b])

    def consume(r, b):
        pltpu.make_async_copy(table_hbm.at[idx_v.at[r]], rows_v.at[b],
                              sems[b]).wait()
        rb = rows_v.at[b]
        for c in range(CHUNKS):
            col = pl.ds(c * LANES, LANES)
            s0 = rb[0, col]
            s1 = rb[1, col]
            for k in range(2, SEQ, 2):
                s0 += rb[k, col]
                s1 += rb[k + 1, col]
            out_v[r, col] = (s0 + s1) * (1.0 / SEQ)

    for b in range(NBUF):
        issue(b, b)

    groups = BPW // NBUF

    def group(g, issue_next):
        for b in range(NBUF):
            r = g * NBUF + b
            consume(r, b)
            if issue_next:
                issue(r + NBUF, b)

    def steady(g, carry):
        group(g, True)
        return carry

    lax.fori_loop(0, groups - 1, steady, 0)
    group(groups - 1, False)

    pltpu.sync_copy(out_v, out_hbm.at[pl.ds(base, BPW)])


_run = pl.kernel(
    _sc_body,
    out_type=jax.ShapeDtypeStruct((BATCH, DIM), jnp.float32),
    mesh=plsc.VectorSubcoreMesh(core_axis_name="c", subcore_axis_name="s",
                                num_cores=NC, num_subcores=NS),
    scratch_types=[
        pltpu.VMEM((BPW, SEQ), jnp.int32),
        pltpu.VMEM((NBUF, SEQ, DIM), jnp.float32),
        pltpu.VMEM((BPW, DIM), jnp.float32),
    ] + [pltpu.SemaphoreType.DMA] * NBUF,
    compiler_params=pltpu.CompilerParams(use_tc_tiling_on_sc=False),
)


def kernel(input_seq_batch, table):
    idx = input_seq_batch.astype(jnp.int32)
    return _run(idx, table)


# speedup vs baseline: 9.0518x; 1.0019x over previous
import jax
import jax.numpy as jnp
from jax import lax
from jax.experimental import pallas as pl
from jax.experimental.pallas import tpu as pltpu
from jax.experimental.pallas import tpu_sc as plsc

BATCH = 4096
SEQ = 50
DIM = 64
NC = 2
NS = 16
NW = NC * NS
BPW = BATCH // NW
NBUF = 4
LANES = 16
CHUNKS = DIM // LANES
SEQP = 56


def _sc_body(idx_hbm, table_hbm, out_hbm, idx_v, rows_v, out_v, *sems):
    wid = lax.axis_index("s") * NC + lax.axis_index("c")
    pltpu.sync_copy(idx_hbm.at[pl.ds(wid * (BPW * SEQP), BPW * SEQP)], idx_v)

    def issue(r, b):
        pltpu.async_copy(table_hbm.at[idx_v.at[pl.ds(r * SEQP, SEQ)]],
                         rows_v.at[b], sems[b])

    def consume(r, b):
        pltpu.make_async_copy(table_hbm.at[idx_v.at[pl.ds(r * SEQP, SEQ)]],
                              rows_v.at[b], sems[b]).wait()
        rb = rows_v.at[b]
        for c in range(CHUNKS):
            col = pl.ds(c * LANES, LANES)
            s0 = rb[0, col]
            s1 = rb[1, col]
            for k in range(2, SEQ, 2):
                s0 += rb[k, col]
                s1 += rb[k + 1, col]
            out_v[pl.ds(r * DIM + c * LANES, LANES)] = (s0 + s1) * (1.0 / SEQ)

    for b in range(NBUF):
        issue(b, b)

    groups = BPW // NBUF

    def group(g, issue_next):
        for b in range(NBUF):
            r = g * NBUF + b
            consume(r, b)
            if issue_next:
                issue(r + NBUF, b)

    def steady(g, carry):
        group(g, True)
        return carry

    lax.fori_loop(0, groups - 1, steady, 0)
    group(groups - 1, False)

    pltpu.sync_copy(out_v, out_hbm.at[pl.ds(wid * (BPW * DIM), BPW * DIM)])


_run = pl.kernel(
    _sc_body,
    out_type=jax.ShapeDtypeStruct((BATCH * DIM,), jnp.float32),
    mesh=plsc.VectorSubcoreMesh(core_axis_name="c", subcore_axis_name="s",
                                num_cores=NC, num_subcores=NS),
    scratch_types=[
        pltpu.VMEM((BPW * SEQP,), jnp.int32),
        pltpu.VMEM((NBUF, SEQ, DIM), jnp.float32),
        pltpu.VMEM((BPW * DIM,), jnp.float32),
    ] + [pltpu.SemaphoreType.DMA] * NBUF,
    compiler_params=pltpu.CompilerParams(use_tc_tiling_on_sc=False),
)


def kernel(input_seq_batch, table):
    idx = jnp.pad(input_seq_batch.astype(jnp.int32),
                  ((0, 0), (0, SEQP - SEQ))).reshape(BATCH * SEQP)
    return _run(idx, table).reshape(BATCH, DIM)


# speedup vs baseline: 11.3565x; 1.2546x over previous
import jax
import jax.numpy as jnp
from jax import lax
from jax.experimental import pallas as pl
from jax.experimental.pallas import tpu as pltpu
from jax.experimental.pallas import tpu_sc as plsc
from jax.experimental import layout as jex_layout

BATCH = 4096
VOCAB = 100000
SEQ = 50
DIM = 64
NC = 2
NS = 16
NW = NC * NS
BPW = BATCH // NW
NBUF = 4
LANES = 16
CHUNKS = DIM // LANES
SEQP = 56


def _sc_body(idx_hbm, table_hbm, out_hbm, idx_v, rows_v, out_v, *sems):
    wid = lax.axis_index("s") * NC + lax.axis_index("c")
    pltpu.sync_copy(idx_hbm.at[pl.ds(wid * (BPW * SEQP), BPW * SEQP)], idx_v)

    def issue(r, b):
        pltpu.async_copy(table_hbm.at[idx_v.at[pl.ds(r * SEQP, SEQ)]],
                         rows_v.at[b], sems[b])

    def consume(r, b):
        pltpu.make_async_copy(table_hbm.at[idx_v.at[pl.ds(r * SEQP, SEQ)]],
                              rows_v.at[b], sems[b]).wait()
        rb = rows_v.at[b]
        for c in range(CHUNKS):
            col = pl.ds(c * LANES, LANES)
            s0 = rb[0, col]
            s1 = rb[1, col]
            for k in range(2, SEQ, 2):
                s0 += rb[k, col]
                s1 += rb[k + 1, col]
            out_v[pl.ds(r * DIM + c * LANES, LANES)] = (s0 + s1) * (1.0 / SEQ)

    for b in range(NBUF):
        issue(b, b)

    groups = BPW // NBUF

    def group(g, issue_next):
        for b in range(NBUF):
            r = g * NBUF + b
            consume(r, b)
            if issue_next:
                issue(r + NBUF, b)

    def steady(g, carry):
        group(g, True)
        return carry

    lax.fori_loop(0, groups - 1, steady, 0)
    group(groups - 1, False)

    pltpu.sync_copy(out_v, out_hbm.at[pl.ds(wid * (BPW * DIM), BPW * DIM)])


_run = pl.kernel(
    _sc_body,
    out_type=jax.ShapeDtypeStruct((BATCH * DIM,), jnp.float32),
    mesh=plsc.VectorSubcoreMesh(core_axis_name="c", subcore_axis_name="s",
                                num_cores=NC, num_subcores=NS),
    scratch_types=[
        pltpu.VMEM((BPW * SEQP,), jnp.int32),
        pltpu.VMEM((NBUF, SEQ, DIM), jnp.float32),
        pltpu.VMEM((BPW * DIM,), jnp.float32),
    ] + [pltpu.SemaphoreType.DMA] * NBUF,
    compiler_params=pltpu.CompilerParams(use_tc_tiling_on_sc=False),
)


def kernel(input_seq_batch, table):
    idx = jnp.pad(input_seq_batch.astype(jnp.int32),
                  ((0, 0), (0, SEQP - SEQ))).reshape(BATCH * SEQP)
    table = jex_layout.with_layout_constraint(
        table, jex_layout.Layout(major_to_minor=(0, 1), tiling=()))
    return _run(idx, table).reshape(BATCH, DIM)
